# baseline (speedup 1.0000x reference)
import functools

import jax
import jax.numpy as jnp
from jax import lax
from jax.experimental import pallas as pl
from jax.experimental.pallas import tpu as pltpu
from jax.experimental.pallas import tpu_sc as plsc

_NC = 25
_B = 1024
_S = 1024
_IH = _B // 8
_JH = _S // 128
_NCHUNK = _IH * _JH
_NW = 32
_CPW = _NCHUNK // _NW
_PLANE = _NCHUNK * 1024


def _sc_call(idx_flat):
    mesh = plsc.VectorSubcoreMesh(core_axis_name="c", subcore_axis_name="s")

    @functools.partial(
        pl.kernel,
        mesh=mesh,
        compiler_params=pltpu.CompilerParams(needs_layout_passes=False),
        out_type=jax.ShapeDtypeStruct((_NC * _PLANE,), jnp.float32),
        scratch_types=[
            pltpu.VMEM((1024,), jnp.int32),
            pltpu.VMEM((_NC * 1024,), jnp.float32),
            pltpu.VMEM((1024,), jnp.int32),
            pltpu.SemaphoreType.DMA,
        ],
    )
    def k(idx_hbm, out_hbm, idx_v, out_v, oldoff_v, sem):
        wid = lax.axis_index("s") * 2 + lax.axis_index("c")
        zeros16f = jnp.zeros((16,), jnp.float32)
        ones16f = jnp.ones((16,), jnp.float32)
        zeros16i = jnp.zeros((16,), jnp.int32)
        iota16 = lax.iota(jnp.int32, 16)

        def zinit(t, _):
            out_v[pl.ds(t * 16, 16)] = zeros16f
            return 0

        lax.fori_loop(0, _NC * 64, zinit, 0)

        def cinit(g, _):
            oldoff_v[pl.ds(g * 16, 16)] = g * 16 + iota16
            return 0

        lax.fori_loop(0, 64, cinit, 0)

        def do_chunk(kk, _):
            chunk = wid * _CPW + kk
            pltpu.sync_copy(idx_hbm.at[pl.ds(chunk * 1024, 1024)], idx_v)

            def group(g, _):
                base = g * 16
                idx16 = idx_v[pl.ds(base, 16)]
                old16 = oldoff_v[pl.ds(base, 16)]
                plsc.store_scatter(out_v, [old16], zeros16f)
                off16 = idx16 * 1024 + (base + iota16)
                oldoff_v[pl.ds(base, 16)] = off16
                plsc.store_scatter(out_v, [off16], ones16f)
                return 0

            lax.fori_loop(0, 64, group, 0)
            copies = [
                pltpu.async_copy(
                    out_v.at[pl.ds(c * 1024, 1024)],
                    out_hbm.at[pl.ds(c * _PLANE + chunk * 1024, 1024)],
                    sem,
                )
                for c in range(_NC)
            ]
            for cp in copies:
                cp.wait()
            return 0

        lax.fori_loop(0, _CPW, do_chunk, 0)

    return k(idx_flat)


def kernel(inputs):
    t = (
        inputs.reshape(_IH, 8, _JH, 128)
        .transpose(0, 2, 1, 3)
        .reshape(_NCHUNK * 1024)
    )
    y = _sc_call(t)
    y5 = y.reshape(_NC, _IH, _JH, 8, 128)
    return y5.transpose(1, 3, 2, 4, 0).reshape(_B, _S, _NC)


# speedup vs baseline: 1.4282x; 1.4282x over previous
import functools

import jax
import jax.numpy as jnp
from jax import lax
from jax.experimental import pallas as pl
from jax.experimental.pallas import tpu as pltpu
from jax.experimental.pallas import tpu_sc as plsc

_NC = 25
_B = 1024
_S = 1024
_IH = _B // 8
_JH = _S // 128
_NCHUNK = _IH * _JH
_NW = 32
_CPW = _NCHUNK // _NW
_PAIR = 2
_NB = _CPW // _PAIR
_W = _PAIR * 1024
_PLANE = _NCHUNK * 1024
_OUTW = _NC * _W


def _sc_call(idx_flat):
    mesh = plsc.VectorSubcoreMesh(core_axis_name="c", subcore_axis_name="s")

    @functools.partial(
        pl.kernel,
        mesh=mesh,
        compiler_params=pltpu.CompilerParams(needs_layout_passes=False),
        out_type=jax.ShapeDtypeStruct((_NC * _PLANE,), jnp.float32),
        scratch_types=[
            pltpu.VMEM((_W,), jnp.int32),
            pltpu.VMEM((_W,), jnp.int32),
            pltpu.VMEM((_OUTW,), jnp.float32),
            pltpu.VMEM((_OUTW,), jnp.float32),
            pltpu.VMEM((_W,), jnp.int32),
            pltpu.VMEM((_W,), jnp.int32),
            pltpu.SemaphoreType.DMA,
            pltpu.SemaphoreType.DMA,
            pltpu.SemaphoreType.DMA,
            pltpu.SemaphoreType.DMA,
        ],
    )
    def k(idx_hbm, out_hbm, i0, i1, o0, o1, f0, f1, si0, si1, so0, so1):
        wid = lax.axis_index("s") * 2 + lax.axis_index("c")
        base_chunk = wid * _CPW
        zeros16f = jnp.zeros((16,), jnp.float32)
        ones16f = jnp.ones((16,), jnp.float32)
        iota16 = lax.iota(jnp.int32, 16)

        idx_v = (i0, i1)
        out_v = (o0, o1)
        off_v = (f0, f1)
        isem = (si0, si1)
        osem = (so0, so1)

        def zinit(t, _):
            o0[pl.ds(t * 16, 16)] = zeros16f
            o1[pl.ds(t * 16, 16)] = zeros16f
            return 0

        lax.fori_loop(0, _OUTW // 16, zinit, 0)

        def cinit(g, _):
            f0[pl.ds(g * 16, 16)] = g * 16 + iota16
            f1[pl.ds(g * 16, 16)] = g * 16 + iota16
            return 0

        lax.fori_loop(0, _W // 16, cinit, 0)

        for b in range(2):
            pltpu.async_copy(
                idx_hbm.at[pl.ds((base_chunk + b * _PAIR) * 1024, _W)],
                idx_v[b], isem[b],
            )

        def run_batch(o, b):
            p = o * 2 + b
            word0 = (base_chunk + p * _PAIR) * 1024
            pltpu.make_async_copy(
                idx_hbm.at[pl.ds(word0, _W)], idx_v[b], isem[b]
            ).wait()

            @pl.when(o >= 1)
            def _():
                pltpu.make_async_copy(
                    out_hbm.at[pl.ds(0, _OUTW)], out_v[b], osem[b]
                ).wait()

            def group(g, _):
                base = g * 16
                idx16 = idx_v[b][pl.ds(base, 16)]
                old16 = off_v[b][pl.ds(base, 16)]
                plsc.store_scatter(out_v[b], [old16], zeros16f)
                off16 = idx16 * _W + (base + iota16)
                off_v[b][pl.ds(base, 16)] = off16
                plsc.store_scatter(out_v[b], [off16], ones16f)
                return 0

            lax.fori_loop(0, _W // 16, group, 0)

            @pl.when(o < _NB // 2 - 1)
            def _():
                pltpu.async_copy(
                    idx_hbm.at[pl.ds(word0 + 2 * _W, _W)], idx_v[b], isem[b]
                )

            for c in range(_NC):
                pltpu.async_copy(
                    out_v[b].at[pl.ds(c * _W, _W)],
                    out_hbm.at[pl.ds(c * _PLANE + word0, _W)],
                    osem[b],
                )
            return 0

        def outer(o, _):
            run_batch(o, 0)
            run_batch(o, 1)
            return 0

        lax.fori_loop(0, _NB // 2, outer, 0)

        for b in range(2):
            pltpu.make_async_copy(
                out_hbm.at[pl.ds(0, _OUTW)], out_v[b], osem[b]
            ).wait()

    return k(idx_flat)


def kernel(inputs):
    t = (
        inputs.reshape(_IH, 8, _JH, 128)
        .transpose(0, 2, 1, 3)
        .reshape(_NCHUNK * 1024)
    )
    y = _sc_call(t)
    y5 = y.reshape(_NC, _IH, _JH, 8, 128)
    return y5.transpose(1, 3, 2, 4, 0).reshape(_B, _S, _NC)
